# baseline (speedup 1.0000x reference)
import functools

import jax
import jax.numpy as jnp
from jax import lax
from jax.experimental import pallas as pl
from jax.experimental.pallas import tpu as pltpu

D_MODEL = 2048
D_SAE = 32768
TOPK = 32
N_ROWS = 32
BS = 1024
N_BLK = D_SAE // BS
CPAD = 128
N_CAND = N_BLK * CPAD

_NEG_INF = float("-inf")


def _local_topk(arr, width):
    iota = lax.broadcasted_iota(jnp.int32, (N_ROWS, width), 1)
    vals, idxs = [], []
    sel = None
    for _ in range(TOPK):
        mx = jnp.max(arr, axis=1, keepdims=True)
        cand = jnp.where(arr == mx, iota, width)
        pos = jnp.min(cand, axis=1, keepdims=True)
        hit = iota == pos
        vals.append(mx)
        idxs.append(pos)
        sel = hit if sel is None else (sel | hit)
        arr = jnp.where(hit, _NEG_INF, arr)
    return (jnp.concatenate(vals, axis=1),
            jnp.concatenate(idxs, axis=1),
            sel)


def _encode_body(x_ref, w_ref, b_ref, out_ref, cv_ref, ci_ref):
    acc = lax.dot_general(
        x_ref[...], w_ref[...],
        dimension_numbers=(((1,), (1,)), ((), ())),
        preferred_element_type=jnp.float32,
    ) + b_ref[...]
    out_ref[...] = acc
    cv, ci, _ = _local_topk(acc, BS)
    cv_ref[...] = jnp.concatenate(
        [cv, jnp.full((N_ROWS, CPAD - TOPK), _NEG_INF, jnp.float32)], axis=1)
    ci_ref[...] = jnp.concatenate(
        [ci, jnp.zeros((N_ROWS, CPAD - TOPK), jnp.int32)], axis=1)


def _decode_body(cv_ref, ci_ref, w_ref, lat_ref, recon_ref,
                 selv_ref, acc_ref):
    j = pl.program_id(0)

    @pl.when(j == 0)
    def _merge():
        _, _, sel = _local_topk(cv_ref[...], N_CAND)
        selv_ref[...] = jnp.where(sel, cv_ref[...], 0.0)
        acc_ref[...] = jnp.zeros_like(acc_ref)

    iota_blk = lax.broadcasted_iota(jnp.int32, (N_ROWS, BS), 1)
    sl = pl.ds(j * CPAD, CPAD)
    v = selv_ref[:, sl][:, :TOPK]
    li = ci_ref[:, sl][:, :TOPK]
    lat = jnp.zeros((N_ROWS, BS), jnp.float32)
    for k in range(TOPK):
        lat = lat + jnp.where(iota_blk == li[:, k:k + 1], v[:, k:k + 1], 0.0)
    lat_ref[...] = lat
    acc_ref[...] += lax.dot_general(
        lat, w_ref[...],
        dimension_numbers=(((1,), (1,)), ((), ())),
        preferred_element_type=jnp.float32,
    )

    @pl.when(j == N_BLK - 1)
    def _emit():
        recon_ref[...] = acc_ref[...]


@jax.jit
def kernel(x, W_enc, b_enc, W_dec):
    b2d = b_enc.reshape(1, D_SAE)

    pre_act, cvals, cidx = pl.pallas_call(
        _encode_body,
        grid=(N_BLK,),
        in_specs=[
            pl.BlockSpec((N_ROWS, D_MODEL), lambda j: (0, 0)),
            pl.BlockSpec((BS, D_MODEL), lambda j: (j, 0)),
            pl.BlockSpec((1, BS), lambda j: (0, j)),
        ],
        out_specs=[
            pl.BlockSpec((N_ROWS, BS), lambda j: (0, j)),
            pl.BlockSpec((N_ROWS, CPAD), lambda j: (0, j)),
            pl.BlockSpec((N_ROWS, CPAD), lambda j: (0, j)),
        ],
        out_shape=[
            jax.ShapeDtypeStruct((N_ROWS, D_SAE), jnp.float32),
            jax.ShapeDtypeStruct((N_ROWS, N_CAND), jnp.float32),
            jax.ShapeDtypeStruct((N_ROWS, N_CAND), jnp.int32),
        ],
    )(x, W_enc, b2d)

    latents, recon = pl.pallas_call(
        _decode_body,
        grid=(N_BLK,),
        in_specs=[
            pl.BlockSpec((N_ROWS, N_CAND), lambda j: (0, 0)),
            pl.BlockSpec((N_ROWS, N_CAND), lambda j: (0, 0)),
            pl.BlockSpec((D_MODEL, BS), lambda j: (0, j)),
        ],
        out_specs=[
            pl.BlockSpec((N_ROWS, BS), lambda j: (0, j)),
            pl.BlockSpec((N_ROWS, D_MODEL), lambda j: (0, 0)),
        ],
        out_shape=[
            jax.ShapeDtypeStruct((N_ROWS, D_SAE), jnp.float32),
            jax.ShapeDtypeStruct((N_ROWS, D_MODEL), jnp.float32),
        ],
        scratch_shapes=[
            pltpu.VMEM((N_ROWS, N_CAND), jnp.float32),
            pltpu.VMEM((N_ROWS, D_MODEL), jnp.float32),
        ],
    )(cvals, cidx, W_dec)

    return recon, latents


# speedup vs baseline: 2.0016x; 2.0016x over previous
import functools

import jax
import jax.numpy as jnp
from jax import lax
from jax.experimental import pallas as pl
from jax.experimental.pallas import tpu as pltpu
from jax.experimental.pallas import tpu_sc as plsc

D_MODEL = 2048
D_SAE = 32768
TOPK = 32
N_ROWS = 32
BS = 1024
N_BLK = D_SAE // BS

_NEG_INF = float("-inf")
_BIG = 2 ** 30

_VPG = 16
_GRP = 16 * _VPG
_NGRP = D_SAE // _GRP
_NSG = _NGRP // 16


def _encode_body(x_ref, w_ref, b_ref, out_ref):
    out_ref[...] = lax.dot_general(
        x_ref[...], w_ref[...],
        dimension_numbers=(((1,), (1,)), ((), ())),
        preferred_element_type=jnp.float32,
    ) + b_ref[...]


def _decode_body(lat_ref, w_ref, recon_ref, acc_ref):
    j = pl.program_id(0)

    @pl.when(j == 0)
    def _init():
        acc_ref[...] = jnp.zeros_like(acc_ref)

    acc_ref[...] += lax.dot_general(
        lat_ref[...], w_ref[...],
        dimension_numbers=(((1,), (1,)), ((), ())),
        preferred_element_type=jnp.float32,
    )

    @pl.when(j == N_BLK - 1)
    def _emit():
        recon_ref[...] = acc_ref[...]


def _shuf(v, idx):
    return lax.gather(
        v, idx[:, None],
        dimension_numbers=lax.GatherDimensionNumbers(
            offset_dims=(), collapsed_slice_dims=(0,), start_index_map=(0,)),
        slice_sizes=(1,),
        mode=lax.GatherScatterMode.PROMISE_IN_BOUNDS)


def _bfly(v, op, lanes):
    for d in (1, 2, 4, 8):
        v = op(v, _shuf(v, lanes ^ d))
    return v


def _sc_topk_body(pre_hbm, lat_hbm, row_v, lat_v, gm_v, sgm_v):
    wid = lax.axis_index("s") * 2 + lax.axis_index("c")
    pltpu.sync_copy(pre_hbm.at[wid], row_v)
    zeros16 = jnp.zeros((16,), jnp.float32)
    lanes = lax.iota(jnp.int32, 16)

    def g_body(g, tok):
        m = row_v[pl.ds(g * _GRP, 16)]
        lat_v[pl.ds(g * _GRP, 16)] = zeros16
        for t in range(1, _VPG):
            m = jnp.maximum(m, row_v[pl.ds(g * _GRP + 16 * t, 16)])
            lat_v[pl.ds(g * _GRP + 16 * t, 16)] = zeros16
        gm_v[pl.ds(g * 16, 16)] = m
        return tok

    lax.fori_loop(0, _NGRP, g_body, 0)

    def s_body(s, tok):
        m = gm_v[pl.ds(s * 256, 16)]
        for t in range(1, 16):
            m = jnp.maximum(m, gm_v[pl.ds(s * 256 + 16 * t, 16)])
        sgm_v[pl.ds(s * 16, 16)] = m
        return tok

    lax.fori_loop(0, _NSG, s_body, 0)

    def ext_body(k, tok):
        best = sgm_v[pl.ds(0, 16)]
        bid = jnp.zeros((16,), jnp.int32)
        for s in range(1, _NSG):
            v = sgm_v[pl.ds(s * 16, 16)]
            c = v > best
            best = jnp.where(c, v, best)
            bid = jnp.where(c, s, bid)
        m_val = _bfly(best, jnp.maximum, lanes)
        sg_v = _bfly(jnp.where(best == m_val, bid, _BIG), jnp.minimum, lanes)
        sg = sg_v[0]

        gfound = jnp.full((16,), _BIG, jnp.int32)
        for t in range(16):
            v = gm_v[pl.ds(sg * 256 + t * 16, 16)]
            gfound = jnp.where((v == m_val) & (gfound == _BIG), t, gfound)
        g_abs = sg * 16 + _bfly(gfound, jnp.minimum, lanes)[0]

        pfound = jnp.full((16,), _BIG, jnp.int32)
        for t in range(_VPG):
            v = row_v[pl.ds(g_abs * _GRP + t * 16, 16)]
            pfound = jnp.where((v == m_val) & (pfound == _BIG), t, pfound)
        p = _bfly(pfound, jnp.minimum, lanes)[0]
        base = g_abs * _GRP + p * 16
        vreg = row_v[pl.ds(base, 16)]
        lane_v = _bfly(jnp.where(vreg == m_val, lanes, _BIG),
                       jnp.minimum, lanes)
        hit = lanes == lane_v
        row_v[pl.ds(base, 16)] = jnp.where(hit, _NEG_INF, vreg)
        lat_v[pl.ds(base, 16)] = jnp.where(hit, m_val, lat_v[pl.ds(base, 16)])

        m = row_v[pl.ds(g_abs * _GRP, 16)]
        for t in range(1, _VPG):
            m = jnp.maximum(m, row_v[pl.ds(g_abs * _GRP + 16 * t, 16)])
        gm_v[pl.ds(g_abs * 16, 16)] = m
        m2 = gm_v[pl.ds(sg * 256, 16)]
        for t in range(1, 16):
            m2 = jnp.maximum(m2, gm_v[pl.ds(sg * 256 + 16 * t, 16)])
        sgm_v[pl.ds(sg * 16, 16)] = m2
        return tok

    lax.fori_loop(0, TOPK, ext_body, 0)
    pltpu.sync_copy(lat_v, lat_hbm.at[wid])


def _sc_topk(pre_act):
    mesh = plsc.VectorSubcoreMesh(core_axis_name="c", subcore_axis_name="s")
    return pl.kernel(
        _sc_topk_body,
        mesh=mesh,
        out_type=jax.ShapeDtypeStruct((N_ROWS, D_SAE), jnp.float32),
        scratch_types=[
            pltpu.VMEM((D_SAE,), jnp.float32),
            pltpu.VMEM((D_SAE,), jnp.float32),
            pltpu.VMEM((_NGRP * 16,), jnp.float32),
            pltpu.VMEM((_NSG * 16,), jnp.float32),
        ],
    )(pre_act)


@jax.jit
def kernel(x, W_enc, b_enc, W_dec):
    b2d = b_enc.reshape(1, D_SAE)

    pre_act = pl.pallas_call(
        _encode_body,
        grid=(N_BLK,),
        in_specs=[
            pl.BlockSpec((N_ROWS, D_MODEL), lambda j: (0, 0)),
            pl.BlockSpec((BS, D_MODEL), lambda j: (j, 0)),
            pl.BlockSpec((1, BS), lambda j: (0, j)),
        ],
        out_specs=pl.BlockSpec((N_ROWS, BS), lambda j: (0, j)),
        out_shape=jax.ShapeDtypeStruct((N_ROWS, D_SAE), jnp.float32),
    )(x, W_enc, b2d)

    latents = _sc_topk(pre_act)

    recon = pl.pallas_call(
        _decode_body,
        grid=(N_BLK,),
        in_specs=[
            pl.BlockSpec((N_ROWS, BS), lambda j: (0, j)),
            pl.BlockSpec((D_MODEL, BS), lambda j: (0, j)),
        ],
        out_specs=pl.BlockSpec((N_ROWS, D_MODEL), lambda j: (0, 0)),
        out_shape=jax.ShapeDtypeStruct((N_ROWS, D_MODEL), jnp.float32),
        scratch_shapes=[pltpu.VMEM((N_ROWS, D_MODEL), jnp.float32)],
    )(latents, W_dec)

    return recon, latents


# speedup vs baseline: 2.0064x; 1.0024x over previous
import functools

import jax
import jax.numpy as jnp
from jax import lax
from jax.experimental import pallas as pl
from jax.experimental.pallas import tpu as pltpu
from jax.experimental.pallas import tpu_sc as plsc

D_MODEL = 2048
D_SAE = 32768
TOPK = 32
N_ROWS = 32
BS = 2048
N_BLK = D_SAE // BS

_NEG_INF = float("-inf")
_BIG = 2 ** 30

_VPG = 16
_GRP = 16 * _VPG
_NGRP = D_SAE // _GRP
_NSG = _NGRP // 16


def _encode_body(x_ref, w_ref, b_ref, out_ref):
    out_ref[...] = lax.dot_general(
        x_ref[...], w_ref[...],
        dimension_numbers=(((1,), (1,)), ((), ())),
        preferred_element_type=jnp.float32,
    ) + b_ref[...]


def _decode_body(lat_ref, w_ref, recon_ref, acc_ref):
    j = pl.program_id(0)

    @pl.when(j == 0)
    def _init():
        acc_ref[...] = jnp.zeros_like(acc_ref)

    acc_ref[...] += lax.dot_general(
        lat_ref[...], w_ref[...],
        dimension_numbers=(((1,), (1,)), ((), ())),
        preferred_element_type=jnp.float32,
    )

    @pl.when(j == N_BLK - 1)
    def _emit():
        recon_ref[...] = acc_ref[...]


def _shuf(v, idx):
    return lax.gather(
        v, idx[:, None],
        dimension_numbers=lax.GatherDimensionNumbers(
            offset_dims=(), collapsed_slice_dims=(0,), start_index_map=(0,)),
        slice_sizes=(1,),
        mode=lax.GatherScatterMode.PROMISE_IN_BOUNDS)


def _bfly(v, op, lanes):
    for d in (1, 2, 4, 8):
        v = op(v, _shuf(v, lanes ^ d))
    return v


def _sc_topk_body(pre_hbm, lat_hbm, row_v, lat_v, gm_v, sgm_v):
    wid = lax.axis_index("s") * 2 + lax.axis_index("c")
    pltpu.sync_copy(pre_hbm.at[wid], row_v)
    zeros16 = jnp.zeros((16,), jnp.float32)
    lanes = lax.iota(jnp.int32, 16)

    def g_body(g, tok):
        m = row_v[pl.ds(g * _GRP, 16)]
        lat_v[pl.ds(g * _GRP, 16)] = zeros16
        for t in range(1, _VPG):
            m = jnp.maximum(m, row_v[pl.ds(g * _GRP + 16 * t, 16)])
            lat_v[pl.ds(g * _GRP + 16 * t, 16)] = zeros16
        gm_v[pl.ds(g * 16, 16)] = m
        return tok

    lax.fori_loop(0, _NGRP, g_body, 0)

    def s_body(s, tok):
        m = gm_v[pl.ds(s * 256, 16)]
        for t in range(1, 16):
            m = jnp.maximum(m, gm_v[pl.ds(s * 256 + 16 * t, 16)])
        sgm_v[pl.ds(s * 16, 16)] = m
        return tok

    lax.fori_loop(0, _NSG, s_body, 0)

    def ext_body(k, tok):
        best = sgm_v[pl.ds(0, 16)]
        bid = jnp.zeros((16,), jnp.int32)
        for s in range(1, _NSG):
            v = sgm_v[pl.ds(s * 16, 16)]
            c = v > best
            best = jnp.where(c, v, best)
            bid = jnp.where(c, s, bid)
        m_val = _bfly(best, jnp.maximum, lanes)
        sg_v = _bfly(jnp.where(best == m_val, bid, _BIG), jnp.minimum, lanes)
        sg = sg_v[0]

        gfound = jnp.full((16,), _BIG, jnp.int32)
        for t in range(16):
            v = gm_v[pl.ds(sg * 256 + t * 16, 16)]
            gfound = jnp.where((v == m_val) & (gfound == _BIG), t, gfound)
        g_abs = sg * 16 + _bfly(gfound, jnp.minimum, lanes)[0]

        pfound = jnp.full((16,), _BIG, jnp.int32)
        for t in range(_VPG):
            v = row_v[pl.ds(g_abs * _GRP + t * 16, 16)]
            pfound = jnp.where((v == m_val) & (pfound == _BIG), t, pfound)
        p = _bfly(pfound, jnp.minimum, lanes)[0]
        base = g_abs * _GRP + p * 16
        vreg = row_v[pl.ds(base, 16)]
        lane_v = _bfly(jnp.where(vreg == m_val, lanes, _BIG),
                       jnp.minimum, lanes)
        hit = lanes == lane_v
        row_v[pl.ds(base, 16)] = jnp.where(hit, _NEG_INF, vreg)
        lat_v[pl.ds(base, 16)] = jnp.where(hit, m_val, lat_v[pl.ds(base, 16)])

        m = row_v[pl.ds(g_abs * _GRP, 16)]
        for t in range(1, _VPG):
            m = jnp.maximum(m, row_v[pl.ds(g_abs * _GRP + 16 * t, 16)])
        gm_v[pl.ds(g_abs * 16, 16)] = m
        m2 = gm_v[pl.ds(sg * 256, 16)]
        for t in range(1, 16):
            m2 = jnp.maximum(m2, gm_v[pl.ds(sg * 256 + 16 * t, 16)])
        sgm_v[pl.ds(sg * 16, 16)] = m2
        return tok

    lax.fori_loop(0, TOPK, ext_body, 0)
    pltpu.sync_copy(lat_v, lat_hbm.at[wid])


def _sc_topk(pre_act):
    mesh = plsc.VectorSubcoreMesh(core_axis_name="c", subcore_axis_name="s")
    return pl.kernel(
        _sc_topk_body,
        mesh=mesh,
        out_type=jax.ShapeDtypeStruct((N_ROWS, D_SAE), jnp.float32),
        scratch_types=[
            pltpu.VMEM((D_SAE,), jnp.float32),
            pltpu.VMEM((D_SAE,), jnp.float32),
            pltpu.VMEM((_NGRP * 16,), jnp.float32),
            pltpu.VMEM((_NSG * 16,), jnp.float32),
        ],
    )(pre_act)


@jax.jit
def kernel(x, W_enc, b_enc, W_dec):
    b2d = b_enc.reshape(1, D_SAE)

    pre_act = pl.pallas_call(
        _encode_body,
        grid=(N_BLK,),
        in_specs=[
            pl.BlockSpec((N_ROWS, D_MODEL), lambda j: (0, 0)),
            pl.BlockSpec((BS, D_MODEL), lambda j: (j, 0)),
            pl.BlockSpec((1, BS), lambda j: (0, j)),
        ],
        out_specs=pl.BlockSpec((N_ROWS, BS), lambda j: (0, j)),
        out_shape=jax.ShapeDtypeStruct((N_ROWS, D_SAE), jnp.float32),
    )(x, W_enc, b2d)

    latents = _sc_topk(pre_act)

    recon = pl.pallas_call(
        _decode_body,
        grid=(N_BLK,),
        in_specs=[
            pl.BlockSpec((N_ROWS, BS), lambda j: (0, j)),
            pl.BlockSpec((D_MODEL, BS), lambda j: (0, j)),
        ],
        out_specs=pl.BlockSpec((N_ROWS, D_MODEL), lambda j: (0, 0)),
        out_shape=jax.ShapeDtypeStruct((N_ROWS, D_MODEL), jnp.float32),
        scratch_shapes=[pltpu.VMEM((N_ROWS, D_MODEL), jnp.float32)],
    )(latents, W_dec)

    return recon, latents
